# baseline (speedup 1.0000x reference)
import jax
import jax.numpy as jnp
from jax import lax
from jax.experimental import pallas as pl
from jax.experimental.pallas import tpu as pltpu
from jax.experimental.pallas import tpu_sc as plsc

N = 10000
D = 128
H = 64
HP = 64
E = 320000
G = 512

NC = 2
NS = 16
NW = NC * NS

C = 80
EPAD = 327680
EPT = EPAD // NW
NCHUNK = EPT // C
BLK = 8
NBLK = NCHUNK // BLK
RING = 2 * BLK
NGB = 8

NPAD = 10240
ROWS_PT = NPAD // NS

_MESH = dict(core_axis_name="c", subcore_axis_name="s",
             num_cores=NC, num_subcores=NS)


def _deg_body(dst_hbm, out_hbm, acc, dst_v, ones_v, zbuf):
    c = lax.axis_index("c")
    s = lax.axis_index("s")
    wid = c * NS + s
    zero16 = jnp.zeros((16,), jnp.float32)
    one16 = jnp.ones((16,), jnp.float32)
    for k in range(ROWS_PT // 16):
        zbuf[pl.ds(16 * k, 16)] = zero16
    for k in range(C // 16):
        ones_v[pl.ds(16 * k, 16)] = one16
    pltpu.sync_copy(zbuf, acc.at[pl.ds(s * ROWS_PT, ROWS_PT)])
    pltpu.sync_copy(dst_hbm.at[wid], dst_v)
    plsc.subcore_barrier()

    def body(j, carry):
        pltpu.sync_copy(ones_v, acc.at[dst_v.at[j]], add=True)
        return carry

    lax.fori_loop(0, NCHUNK, body, 0)
    plsc.subcore_barrier()
    pltpu.sync_copy(acc.at[pl.ds(s * ROWS_PT, ROWS_PT)],
                    out_hbm.at[c, pl.ds(s * ROWS_PT, ROWS_PT)])


def _deg_kernel(dst3):
    k = pl.kernel(
        _deg_body,
        out_type=jax.ShapeDtypeStruct((NC, NPAD), jnp.float32),
        mesh=plsc.VectorSubcoreMesh(**_MESH),
        compiler_params=pltpu.CompilerParams(use_tc_tiling_on_sc=False),
        scratch_types=[
            pltpu.VMEM_SHARED((NPAD,), jnp.float32),
            pltpu.VMEM((NCHUNK, C), jnp.int32),
            pltpu.VMEM((C,), jnp.float32),
            pltpu.VMEM((ROWS_PT,), jnp.float32),
        ],
    )
    return k(dst3)


_ZROWS = 32


def _mp_body(u_hbm, src_hbm, dst_hbm, out_hbm,
             acc, srcb, dstb, gbufs, zbuf, sem_i, gsems, semz):
    c = lax.axis_index("c")
    s = lax.axis_index("s")
    wid = c * NS + s

    zero16 = jnp.zeros((16,), jnp.float32)
    for i in range(_ZROWS):
        for k in range(HP // 16):
            zbuf[i, pl.ds(16 * k, 16)] = zero16
    for t in range(ROWS_PT // _ZROWS):
        pltpu.async_copy(
            zbuf, acc.at[pl.ds(s * ROWS_PT + t * _ZROWS, _ZROWS)], semz)
    for t in range(ROWS_PT // _ZROWS):
        pltpu.make_async_copy(
            zbuf, acc.at[pl.ds(s * ROWS_PT + t * _ZROWS, _ZROWS)],
            semz).wait()
    plsc.subcore_barrier()

    def prefetch_blk(b):
        slot = lax.rem(b, 2) * BLK
        pltpu.async_copy(src_hbm.at[wid, pl.ds(b * BLK, BLK)],
                         srcb.at[pl.ds(slot, BLK)], sem_i)
        pltpu.async_copy(dst_hbm.at[wid, pl.ds(b * BLK, BLK)],
                         dstb.at[pl.ds(slot, BLK)], sem_i)

    def wait_blk(b):
        slot = lax.rem(b, 2) * BLK
        pltpu.make_async_copy(src_hbm.at[wid, pl.ds(b * BLK, BLK)],
                              srcb.at[pl.ds(slot, BLK)], sem_i).wait()
        pltpu.make_async_copy(dst_hbm.at[wid, pl.ds(b * BLK, BLK)],
                              dstb.at[pl.ds(slot, BLK)], sem_i).wait()

    def fire_gather(j, k):
        pltpu.async_copy(u_hbm.at[srcb.at[lax.rem(j, RING)]],
                         gbufs[k], gsems[k])

    def drain_scatter(j, k):
        pltpu.make_async_copy(
            u_hbm.at[srcb.at[lax.rem(j, RING)]], gbufs[k], gsems[k]).wait()
        pltpu.sync_copy(gbufs[k], acc.at[dstb.at[lax.rem(j, RING)]],
                        add=True)

    prefetch_blk(jnp.int32(0))
    prefetch_blk(jnp.int32(1))
    wait_blk(jnp.int32(0))
    for k in range(NGB - 1):
        fire_gather(jnp.int32(k), k)

    def body(b, carry):
        j = b * BLK

        @pl.when(b + 1 < NBLK)
        def _():
            wait_blk(b + 1)

        for jo in range(BLK):
            @pl.when(j + jo + NGB - 1 < NCHUNK)
            def _():
                fire_gather(j + jo + NGB - 1, (jo + NGB - 1) % NGB)

            drain_scatter(j + jo, jo % NGB)

        @pl.when(b + 2 < NBLK)
        def _():
            prefetch_blk(b + 2)

        return carry

    lax.fori_loop(0, NBLK, body, 0)
    plsc.subcore_barrier()
    pltpu.sync_copy(acc.at[pl.ds(s * ROWS_PT, ROWS_PT)],
                    out_hbm.at[c, pl.ds(s * ROWS_PT, ROWS_PT)])


def _mp_kernel(u, src3, dst3):
    def body(u_hbm, src_hbm, dst_hbm, out_hbm, acc, srcb, dstb,
             g0, g1, g2, g3, g4, g5, g6, g7, zbuf, sem_i,
             s0, s1, s2, s3, s4, s5, s6, s7, semz):
        _mp_body(u_hbm, src_hbm, dst_hbm, out_hbm, acc, srcb, dstb,
                 [g0, g1, g2, g3, g4, g5, g6, g7], zbuf, sem_i,
                 [s0, s1, s2, s3, s4, s5, s6, s7], semz)

    k = pl.kernel(
        body,
        out_type=jax.ShapeDtypeStruct((NC, NPAD, HP), jnp.float32),
        mesh=plsc.VectorSubcoreMesh(**_MESH),
        compiler_params=pltpu.CompilerParams(use_tc_tiling_on_sc=False),
        scratch_types=[
            pltpu.VMEM_SHARED((NPAD, HP), jnp.float32),
            pltpu.VMEM((RING, C), jnp.int32),
            pltpu.VMEM((RING, C), jnp.int32),
        ] + [pltpu.VMEM((C, HP), jnp.float32)] * NGB + [
            pltpu.VMEM((_ZROWS, HP), jnp.float32),
            pltpu.SemaphoreType.DMA,
        ] + [pltpu.SemaphoreType.DMA] * NGB + [
            pltpu.SemaphoreType.DMA,
        ],
    )
    return k(u, src3, dst3)


def _row_mask():
    rows = lax.broadcasted_iota(jnp.int32, (NPAD, 1), 0)
    return rows < N


def _ka_body(degp_ref, x_ref, w_ref, u_ref, dinv_ref):
    p = degp_ref[...]
    deg = p[0] + p[1] + 1.0
    dinv = lax.rsqrt(jnp.maximum(deg, 1.0))
    dinv_ref[...] = dinv
    h = jnp.dot(x_ref[...], w_ref[...], preferred_element_type=jnp.float32)
    u_ref[...] = h * dinv


def _ka_kernel(degp, x_pad, W1p):
    degp3 = degp.reshape(NC, NPAD, 1)
    return pl.pallas_call(
        _ka_body,
        out_shape=(jax.ShapeDtypeStruct((NPAD, HP), jnp.float32),
                   jax.ShapeDtypeStruct((NPAD, 1), jnp.float32)),
    )(degp3, x_pad, W1p)


def _bn_relu_masked(agg, gamma, beta, mask):
    aggm = jnp.where(mask, agg, 0.0)
    mu = jnp.sum(aggm, axis=0, keepdims=True) * (1.0 / N)
    cen = jnp.where(mask, agg - mu, 0.0)
    var = jnp.sum(cen * cen, axis=0, keepdims=True) * (1.0 / N)
    z = cen * lax.rsqrt(var + 1e-5) * gamma + beta
    return jnp.where(mask, jnp.maximum(z, 0.0), 0.0)


def _kb_body(p_ref, u_ref, dinv_ref, b_ref, g_ref, be_ref, wn_ref, un_ref):
    p = p_ref[...]
    mask = _row_mask()
    agg = dinv_ref[...] * (p[0] + p[1] + u_ref[...]) + b_ref[...]
    z = _bn_relu_masked(agg, g_ref[...], be_ref[...], mask)
    un_ref[...] = jnp.dot(z, wn_ref[...],
                          preferred_element_type=jnp.float32) * dinv_ref[...]


def _kb_kernel(p, u, dinv_col, bp, gp, bep, Wnp):
    return pl.pallas_call(
        _kb_body,
        out_shape=jax.ShapeDtypeStruct((NPAD, HP), jnp.float32),
    )(p, u, dinv_col, bp, gp, bep, Wnp)


_POOL_CHUNK = 1024


def _kc_body(p_ref, u_ref, dinv_ref, b_ref, g_ref, be_ref, batch_ref,
             fw1_ref, fb1_ref, fw2_ref, fb2_ref, out_ref):
    p = p_ref[...]
    mask = _row_mask()
    agg = dinv_ref[...] * (p[0] + p[1] + u_ref[...]) + b_ref[...]
    z = _bn_relu_masked(agg, g_ref[...], be_ref[...], mask)

    batch = batch_ref[...]
    sums = jnp.zeros((G, HP), jnp.float32)
    cnt = jnp.zeros((G, 1), jnp.float32)
    dn = (((0,), (0,)), ((), ()))
    ids = lax.broadcasted_iota(jnp.int32, (_POOL_CHUNK, G), 1)
    ones_col = jnp.ones((_POOL_CHUNK, 1), jnp.float32)
    for i in range(NPAD // _POOL_CHUNK):
        zc = z[i * _POOL_CHUNK:(i + 1) * _POOL_CHUNK]
        bc = batch[i * _POOL_CHUNK:(i + 1) * _POOL_CHUNK]
        onehot = (ids == bc).astype(jnp.float32)
        sums = sums + lax.dot_general(onehot, zc, dn,
                                      preferred_element_type=jnp.float32)
        cnt = cnt + lax.dot_general(onehot, ones_col, dn,
                                    preferred_element_type=jnp.float32)
    pooled = sums / jnp.maximum(cnt, 1.0)
    hfc = jnp.maximum(
        jnp.dot(pooled, fw1_ref[...], preferred_element_type=jnp.float32)
        + fb1_ref[...], 0.0)
    out_ref[...] = (jnp.dot(hfc, fw2_ref[...],
                            preferred_element_type=jnp.float32)
                    + fb2_ref[...])


def _kc_kernel(p, u, dinv_col, bp, gp, bep, batch_pad, fW1p, fb1, fW2, fb2):
    out = pl.pallas_call(
        _kc_body,
        out_shape=jax.ShapeDtypeStruct((G, 1), jnp.float32),
    )(p, u, dinv_col, bp, gp, bep,
      batch_pad, fW1p, fb1.reshape(1, 32), fW2, fb2.reshape(1, 1))
    return out.reshape(G)


def _pad_cols(a):
    return jnp.pad(a, ((0, 0), (0, HP - H)))


def kernel(x, edge_index, batch, W1, b1, W2, b2, W3, b3,
           g1, be1, g2, be2, g3, be3, fW1, fb1, fW2, fb2):
    pad_e = N + jnp.arange(EPAD - E, dtype=jnp.int32) % (NPAD - N)
    src3 = jnp.concatenate([edge_index[0], pad_e]).reshape(NW, NCHUNK, C)
    dst3 = jnp.concatenate([edge_index[1], pad_e]).reshape(NW, NCHUNK, C)
    x_pad = jnp.pad(x, ((0, NPAD - N), (0, 0)))
    batch_pad = jnp.pad(batch, (0, NPAD - N),
                        constant_values=G).reshape(NPAD, 1)

    W1p = _pad_cols(W1)
    W2p = _pad_cols(jnp.pad(W2, ((0, HP - H), (0, 0))))
    W3p = _pad_cols(jnp.pad(W3, ((0, HP - H), (0, 0))))
    fW1p = jnp.pad(fW1, ((0, HP - H), (0, 0)))
    b1p = _pad_cols(b1.reshape(1, H))
    b2p = _pad_cols(b2.reshape(1, H))
    b3p = _pad_cols(b3.reshape(1, H))
    g1p = _pad_cols(g1.reshape(1, H))
    g2p = _pad_cols(g2.reshape(1, H))
    g3p = _pad_cols(g3.reshape(1, H))
    be1p = _pad_cols(be1.reshape(1, H))
    be2p = _pad_cols(be2.reshape(1, H))
    be3p = _pad_cols(be3.reshape(1, H))

    degp = _deg_kernel(dst3)
    u1, dinv_col = _ka_kernel(degp, x_pad, W1p)
    p1 = _mp_kernel(u1, src3, dst3)
    u2 = _kb_kernel(p1, u1, dinv_col, b1p, g1p, be1p, W2p)
    p2 = _mp_kernel(u2, src3, dst3)
    u3 = _kb_kernel(p2, u2, dinv_col, b2p, g2p, be2p, W3p)
    p3 = _mp_kernel(u3, src3, dst3)
    return _kc_kernel(p3, u3, dinv_col, b3p, g3p, be3p, batch_pad,
                      fW1p, fb1, fW2, fb2)


# speedup vs baseline: 1.0672x; 1.0672x over previous
import jax
import jax.numpy as jnp
from jax import lax
from jax.experimental import pallas as pl
from jax.experimental.pallas import tpu as pltpu
from jax.experimental.pallas import tpu_sc as plsc

N = 10000
D = 128
H = 64
HP = 64
E = 320000
G = 512

NC = 2
NS = 16
NW = NC * NS

C = 128
EPAD = 327680
EPT = EPAD // NW
NCHUNK = EPT // C
BLK = 8
NBLK = NCHUNK // BLK
RING = 2 * BLK
NGB = 8

NPAD = 10240
ROWS_PT = NPAD // NS

_MESH = dict(core_axis_name="c", subcore_axis_name="s",
             num_cores=NC, num_subcores=NS)


def _deg_body(dst_hbm, out_hbm, acc, dst_v, ones_v, zbuf):
    c = lax.axis_index("c")
    s = lax.axis_index("s")
    wid = c * NS + s
    zero16 = jnp.zeros((16,), jnp.float32)
    one16 = jnp.ones((16,), jnp.float32)
    for k in range(ROWS_PT // 16):
        zbuf[pl.ds(16 * k, 16)] = zero16
    for k in range(C // 16):
        ones_v[pl.ds(16 * k, 16)] = one16
    pltpu.sync_copy(zbuf, acc.at[pl.ds(s * ROWS_PT, ROWS_PT)])
    pltpu.sync_copy(dst_hbm.at[wid], dst_v)
    plsc.subcore_barrier()

    def body(j, carry):
        pltpu.sync_copy(ones_v, acc.at[dst_v.at[j]], add=True)
        return carry

    lax.fori_loop(0, NCHUNK, body, 0)
    plsc.subcore_barrier()
    pltpu.sync_copy(acc.at[pl.ds(s * ROWS_PT, ROWS_PT)],
                    out_hbm.at[c, pl.ds(s * ROWS_PT, ROWS_PT)])


def _deg_kernel(dst3):
    k = pl.kernel(
        _deg_body,
        out_type=jax.ShapeDtypeStruct((NC, NPAD), jnp.float32),
        mesh=plsc.VectorSubcoreMesh(**_MESH),
        compiler_params=pltpu.CompilerParams(use_tc_tiling_on_sc=False),
        scratch_types=[
            pltpu.VMEM_SHARED((NPAD,), jnp.float32),
            pltpu.VMEM((NCHUNK, C), jnp.int32),
            pltpu.VMEM((C,), jnp.float32),
            pltpu.VMEM((ROWS_PT,), jnp.float32),
        ],
    )
    return k(dst3)


_ZROWS = 32


def _mp_body(u_hbm, src_hbm, dst_hbm, out_hbm,
             acc, srcb, dstb, gbufs, zbuf, sem_i, gsems, semz):
    c = lax.axis_index("c")
    s = lax.axis_index("s")
    wid = c * NS + s

    zero16 = jnp.zeros((16,), jnp.float32)
    for i in range(_ZROWS):
        for k in range(HP // 16):
            zbuf[i, pl.ds(16 * k, 16)] = zero16
    for t in range(ROWS_PT // _ZROWS):
        pltpu.async_copy(
            zbuf, acc.at[pl.ds(s * ROWS_PT + t * _ZROWS, _ZROWS)], semz)
    for t in range(ROWS_PT // _ZROWS):
        pltpu.make_async_copy(
            zbuf, acc.at[pl.ds(s * ROWS_PT + t * _ZROWS, _ZROWS)],
            semz).wait()
    plsc.subcore_barrier()

    def prefetch_blk(b):
        slot = lax.rem(b, 2) * BLK
        pltpu.async_copy(src_hbm.at[wid, pl.ds(b * BLK, BLK)],
                         srcb.at[pl.ds(slot, BLK)], sem_i)
        pltpu.async_copy(dst_hbm.at[wid, pl.ds(b * BLK, BLK)],
                         dstb.at[pl.ds(slot, BLK)], sem_i)

    def wait_blk(b):
        slot = lax.rem(b, 2) * BLK
        pltpu.make_async_copy(src_hbm.at[wid, pl.ds(b * BLK, BLK)],
                              srcb.at[pl.ds(slot, BLK)], sem_i).wait()
        pltpu.make_async_copy(dst_hbm.at[wid, pl.ds(b * BLK, BLK)],
                              dstb.at[pl.ds(slot, BLK)], sem_i).wait()

    def fire_gather(j, k):
        pltpu.async_copy(u_hbm.at[srcb.at[lax.rem(j, RING)]],
                         gbufs[k], gsems[k])

    def drain_scatter(j, k):
        pltpu.make_async_copy(
            u_hbm.at[srcb.at[lax.rem(j, RING)]], gbufs[k], gsems[k]).wait()
        pltpu.sync_copy(gbufs[k], acc.at[dstb.at[lax.rem(j, RING)]],
                        add=True)

    prefetch_blk(jnp.int32(0))
    prefetch_blk(jnp.int32(1))
    wait_blk(jnp.int32(0))
    for k in range(NGB - 1):
        fire_gather(jnp.int32(k), k)

    def body(b, carry):
        j = b * BLK

        @pl.when(b + 1 < NBLK)
        def _():
            wait_blk(b + 1)

        for jo in range(BLK):
            @pl.when(j + jo + NGB - 1 < NCHUNK)
            def _():
                fire_gather(j + jo + NGB - 1, (jo + NGB - 1) % NGB)

            drain_scatter(j + jo, jo % NGB)

        @pl.when(b + 2 < NBLK)
        def _():
            prefetch_blk(b + 2)

        return carry

    lax.fori_loop(0, NBLK, body, 0)
    plsc.subcore_barrier()
    pltpu.sync_copy(acc.at[pl.ds(s * ROWS_PT, ROWS_PT)],
                    out_hbm.at[c, pl.ds(s * ROWS_PT, ROWS_PT)])


def _mp_kernel(u, src3, dst3):
    def body(u_hbm, src_hbm, dst_hbm, out_hbm, acc, srcb, dstb,
             g0, g1, g2, g3, g4, g5, g6, g7, zbuf, sem_i,
             s0, s1, s2, s3, s4, s5, s6, s7, semz):
        _mp_body(u_hbm, src_hbm, dst_hbm, out_hbm, acc, srcb, dstb,
                 [g0, g1, g2, g3, g4, g5, g6, g7], zbuf, sem_i,
                 [s0, s1, s2, s3, s4, s5, s6, s7], semz)

    k = pl.kernel(
        body,
        out_type=jax.ShapeDtypeStruct((NC, NPAD, HP), jnp.float32),
        mesh=plsc.VectorSubcoreMesh(**_MESH),
        compiler_params=pltpu.CompilerParams(use_tc_tiling_on_sc=False),
        scratch_types=[
            pltpu.VMEM_SHARED((NPAD, HP), jnp.float32),
            pltpu.VMEM((RING, C), jnp.int32),
            pltpu.VMEM((RING, C), jnp.int32),
        ] + [pltpu.VMEM((C, HP), jnp.float32)] * NGB + [
            pltpu.VMEM((_ZROWS, HP), jnp.float32),
            pltpu.SemaphoreType.DMA,
        ] + [pltpu.SemaphoreType.DMA] * NGB + [
            pltpu.SemaphoreType.DMA,
        ],
    )
    return k(u, src3, dst3)


def _row_mask():
    rows = lax.broadcasted_iota(jnp.int32, (NPAD, 1), 0)
    return rows < N


def _fin_body(degp_ref, dinv_ref):
    p = degp_ref[...]
    deg = p[0] + p[1] + 1.0
    dinv_ref[...] = lax.rsqrt(jnp.maximum(deg, 1.0))


def _fin_kernel(degp):
    degp3 = degp.reshape(NC, NPAD // 128, 128)
    out = pl.pallas_call(
        _fin_body,
        out_shape=jax.ShapeDtypeStruct((NPAD // 128, 128), jnp.float32),
    )(degp3)
    return out.reshape(NPAD, 1)


def _ka_body(x_ref, w_ref, dinv_ref, u_ref):
    h = jnp.dot(x_ref[...], w_ref[...], preferred_element_type=jnp.float32)
    u_ref[...] = h * dinv_ref[...]


def _ka_kernel(x_pad, W1p, dinv_col):
    return pl.pallas_call(
        _ka_body,
        out_shape=jax.ShapeDtypeStruct((NPAD, HP), jnp.float32),
    )(x_pad, W1p, dinv_col)


def _bn_relu_masked(agg, gamma, beta, mask):
    aggm = jnp.where(mask, agg, 0.0)
    mu = jnp.sum(aggm, axis=0, keepdims=True) * (1.0 / N)
    cen = jnp.where(mask, agg - mu, 0.0)
    var = jnp.sum(cen * cen, axis=0, keepdims=True) * (1.0 / N)
    z = cen * lax.rsqrt(var + 1e-5) * gamma + beta
    return jnp.where(mask, jnp.maximum(z, 0.0), 0.0)


def _kb_body(p_ref, u_ref, dinv_ref, b_ref, g_ref, be_ref, wn_ref, un_ref):
    p = p_ref[...]
    mask = _row_mask()
    agg = dinv_ref[...] * (p[0] + p[1] + u_ref[...]) + b_ref[...]
    z = _bn_relu_masked(agg, g_ref[...], be_ref[...], mask)
    un_ref[...] = jnp.dot(z, wn_ref[...],
                          preferred_element_type=jnp.float32) * dinv_ref[...]


def _kb_kernel(p, u, dinv_col, bp, gp, bep, Wnp):
    return pl.pallas_call(
        _kb_body,
        out_shape=jax.ShapeDtypeStruct((NPAD, HP), jnp.float32),
    )(p, u, dinv_col, bp, gp, bep, Wnp)


_POOL_CHUNK = 1024


def _kc_body(p_ref, u_ref, dinv_ref, b_ref, g_ref, be_ref, batch_ref,
             fw1_ref, fb1_ref, fw2_ref, fb2_ref, out_ref):
    p = p_ref[...]
    mask = _row_mask()
    agg = dinv_ref[...] * (p[0] + p[1] + u_ref[...]) + b_ref[...]
    z = _bn_relu_masked(agg, g_ref[...], be_ref[...], mask)

    batch = batch_ref[...]
    sums = jnp.zeros((G, HP), jnp.float32)
    cnt = jnp.zeros((G, 1), jnp.float32)
    dn = (((0,), (0,)), ((), ()))
    ids = lax.broadcasted_iota(jnp.int32, (_POOL_CHUNK, G), 1)
    ones_col = jnp.ones((_POOL_CHUNK, 1), jnp.float32)
    for i in range(NPAD // _POOL_CHUNK):
        zc = z[i * _POOL_CHUNK:(i + 1) * _POOL_CHUNK]
        bc = batch[i * _POOL_CHUNK:(i + 1) * _POOL_CHUNK]
        onehot = (ids == bc).astype(jnp.float32)
        sums = sums + lax.dot_general(onehot, zc, dn,
                                      preferred_element_type=jnp.float32)
        cnt = cnt + lax.dot_general(onehot, ones_col, dn,
                                    preferred_element_type=jnp.float32)
    pooled = sums / jnp.maximum(cnt, 1.0)
    hfc = jnp.maximum(
        jnp.dot(pooled, fw1_ref[...], preferred_element_type=jnp.float32)
        + fb1_ref[...], 0.0)
    out_ref[...] = (jnp.dot(hfc, fw2_ref[...],
                            preferred_element_type=jnp.float32)
                    + fb2_ref[...])


def _kc_kernel(p, u, dinv_col, bp, gp, bep, batch_pad, fW1p, fb1, fW2, fb2):
    out = pl.pallas_call(
        _kc_body,
        out_shape=jax.ShapeDtypeStruct((G, 1), jnp.float32),
    )(p, u, dinv_col, bp, gp, bep,
      batch_pad, fW1p, fb1.reshape(1, 32), fW2, fb2.reshape(1, 1))
    return out.reshape(G)


def _pad_cols(a):
    return jnp.pad(a, ((0, 0), (0, HP - H)))


def kernel(x, edge_index, batch, W1, b1, W2, b2, W3, b3,
           g1, be1, g2, be2, g3, be3, fW1, fb1, fW2, fb2):
    pad_e = N + jnp.arange(EPAD - E, dtype=jnp.int32) % (NPAD - N)
    src3 = jnp.concatenate([edge_index[0], pad_e]).reshape(NW, NCHUNK, C)
    dst3 = jnp.concatenate([edge_index[1], pad_e]).reshape(NW, NCHUNK, C)
    x_pad = jnp.pad(x, ((0, NPAD - N), (0, 0)))
    batch_pad = jnp.pad(batch, (0, NPAD - N),
                        constant_values=G).reshape(NPAD, 1)

    W1p = _pad_cols(W1)
    W2p = _pad_cols(jnp.pad(W2, ((0, HP - H), (0, 0))))
    W3p = _pad_cols(jnp.pad(W3, ((0, HP - H), (0, 0))))
    fW1p = jnp.pad(fW1, ((0, HP - H), (0, 0)))
    b1p = _pad_cols(b1.reshape(1, H))
    b2p = _pad_cols(b2.reshape(1, H))
    b3p = _pad_cols(b3.reshape(1, H))
    g1p = _pad_cols(g1.reshape(1, H))
    g2p = _pad_cols(g2.reshape(1, H))
    g3p = _pad_cols(g3.reshape(1, H))
    be1p = _pad_cols(be1.reshape(1, H))
    be2p = _pad_cols(be2.reshape(1, H))
    be3p = _pad_cols(be3.reshape(1, H))

    degp = _deg_kernel(dst3)
    dinv_col = _fin_kernel(degp)
    u1 = _ka_kernel(x_pad, W1p, dinv_col)
    p1 = _mp_kernel(u1, src3, dst3)
    u2 = _kb_kernel(p1, u1, dinv_col, b1p, g1p, be1p, W2p)
    p2 = _mp_kernel(u2, src3, dst3)
    u3 = _kb_kernel(p2, u2, dinv_col, b2p, g2p, be2p, W3p)
    p3 = _mp_kernel(u3, src3, dst3)
    return _kc_kernel(p3, u3, dinv_col, b3p, g3p, be3p, batch_pad,
                      fW1p, fb1, fW2, fb2)


# speedup vs baseline: 1.0785x; 1.0106x over previous
import jax
import jax.numpy as jnp
from jax import lax
from jax.experimental import pallas as pl
from jax.experimental.pallas import tpu as pltpu
from jax.experimental.pallas import tpu_sc as plsc

N = 10000
D = 128
H = 64
HP = 64
E = 320000
G = 512

NC = 2
NS = 16
NW = NC * NS

C = 128
EPAD = 327680
EPT = EPAD // NW
NCHUNK = EPT // C
BLK = 8
NBLK = NCHUNK // BLK
RING = 2 * BLK
NGB = 8

NPAD = 10240
ROWS_PT = NPAD // NS

_MESH = dict(core_axis_name="c", subcore_axis_name="s",
             num_cores=NC, num_subcores=NS)


def _deg_body(dst_hbm, out_hbm, acc, dst_v, ones_v, zbuf, sem_d):
    c = lax.axis_index("c")
    s = lax.axis_index("s")
    wid = c * NS + s
    zero16 = jnp.zeros((16,), jnp.float32)
    one16 = jnp.ones((16,), jnp.float32)
    for k in range(ROWS_PT // 16):
        zbuf[pl.ds(16 * k, 16)] = zero16
    for k in range(C // 16):
        ones_v[pl.ds(16 * k, 16)] = one16
    pltpu.sync_copy(zbuf, acc.at[pl.ds(s * ROWS_PT, ROWS_PT)])
    pltpu.sync_copy(dst_hbm.at[wid], dst_v)
    plsc.subcore_barrier()

    _DEG_WIN = 20
    for j in range(NCHUNK):
        pltpu.async_copy(ones_v, acc.at[dst_v.at[j]], sem_d, add=True)
        if j >= _DEG_WIN:
            pltpu.make_async_copy(
                ones_v, acc.at[dst_v.at[j - _DEG_WIN]], sem_d).wait()
    for j in range(NCHUNK - _DEG_WIN, NCHUNK):
        pltpu.make_async_copy(ones_v, acc.at[dst_v.at[j]], sem_d).wait()
    plsc.subcore_barrier()
    pltpu.sync_copy(acc.at[pl.ds(s * ROWS_PT, ROWS_PT)],
                    out_hbm.at[c, pl.ds(s * ROWS_PT, ROWS_PT)])


def _deg_kernel(dst3):
    k = pl.kernel(
        _deg_body,
        out_type=jax.ShapeDtypeStruct((NC, NPAD), jnp.float32),
        mesh=plsc.VectorSubcoreMesh(**_MESH),
        compiler_params=pltpu.CompilerParams(use_tc_tiling_on_sc=False),
        scratch_types=[
            pltpu.VMEM_SHARED((NPAD,), jnp.float32),
            pltpu.VMEM((NCHUNK, C), jnp.int32),
            pltpu.VMEM((C,), jnp.float32),
            pltpu.VMEM((ROWS_PT,), jnp.float32),
            pltpu.SemaphoreType.DMA,
        ],
    )
    return k(dst3)


_ZROWS = 32


def _mp_body(u_hbm, src_hbm, dst_hbm, out_hbm,
             acc, srcb, dstb, gbufs, zbuf, sem_i, gsems, semz):
    c = lax.axis_index("c")
    s = lax.axis_index("s")
    wid = c * NS + s

    zero16 = jnp.zeros((16,), jnp.float32)
    for i in range(_ZROWS):
        for k in range(HP // 16):
            zbuf[i, pl.ds(16 * k, 16)] = zero16
    for t in range(ROWS_PT // _ZROWS):
        pltpu.async_copy(
            zbuf, acc.at[pl.ds(s * ROWS_PT + t * _ZROWS, _ZROWS)], semz)
    for t in range(ROWS_PT // _ZROWS):
        pltpu.make_async_copy(
            zbuf, acc.at[pl.ds(s * ROWS_PT + t * _ZROWS, _ZROWS)],
            semz).wait()
    plsc.subcore_barrier()

    def prefetch_blk(b):
        slot = lax.rem(b, 2) * BLK
        pltpu.async_copy(src_hbm.at[wid, pl.ds(b * BLK, BLK)],
                         srcb.at[pl.ds(slot, BLK)], sem_i)
        pltpu.async_copy(dst_hbm.at[wid, pl.ds(b * BLK, BLK)],
                         dstb.at[pl.ds(slot, BLK)], sem_i)

    def wait_blk(b):
        slot = lax.rem(b, 2) * BLK
        pltpu.make_async_copy(src_hbm.at[wid, pl.ds(b * BLK, BLK)],
                              srcb.at[pl.ds(slot, BLK)], sem_i).wait()
        pltpu.make_async_copy(dst_hbm.at[wid, pl.ds(b * BLK, BLK)],
                              dstb.at[pl.ds(slot, BLK)], sem_i).wait()

    def fire_gather(j, k):
        pltpu.async_copy(u_hbm.at[srcb.at[lax.rem(j, RING)]],
                         gbufs[k], gsems[k])

    def drain_scatter(j, k):
        pltpu.make_async_copy(
            u_hbm.at[srcb.at[lax.rem(j, RING)]], gbufs[k], gsems[k]).wait()
        pltpu.sync_copy(gbufs[k], acc.at[dstb.at[lax.rem(j, RING)]],
                        add=True)

    prefetch_blk(jnp.int32(0))
    prefetch_blk(jnp.int32(1))
    wait_blk(jnp.int32(0))
    for k in range(NGB - 1):
        fire_gather(jnp.int32(k), k)

    def body(b, carry):
        j = b * BLK

        @pl.when(b + 1 < NBLK)
        def _():
            wait_blk(b + 1)

        for jo in range(BLK):
            @pl.when(j + jo + NGB - 1 < NCHUNK)
            def _():
                fire_gather(j + jo + NGB - 1, (jo + NGB - 1) % NGB)

            drain_scatter(j + jo, jo % NGB)

        @pl.when(b + 2 < NBLK)
        def _():
            prefetch_blk(b + 2)

        return carry

    lax.fori_loop(0, NBLK, body, 0)
    plsc.subcore_barrier()
    pltpu.sync_copy(acc.at[pl.ds(s * ROWS_PT, ROWS_PT)],
                    out_hbm.at[c, pl.ds(s * ROWS_PT, ROWS_PT)])


def _mp_kernel(u, src3, dst3):
    def body(u_hbm, src_hbm, dst_hbm, out_hbm, acc, srcb, dstb,
             g0, g1, g2, g3, g4, g5, g6, g7, zbuf, sem_i,
             s0, s1, s2, s3, s4, s5, s6, s7, semz):
        _mp_body(u_hbm, src_hbm, dst_hbm, out_hbm, acc, srcb, dstb,
                 [g0, g1, g2, g3, g4, g5, g6, g7], zbuf, sem_i,
                 [s0, s1, s2, s3, s4, s5, s6, s7], semz)

    k = pl.kernel(
        body,
        out_type=jax.ShapeDtypeStruct((NC, NPAD, HP), jnp.float32),
        mesh=plsc.VectorSubcoreMesh(**_MESH),
        compiler_params=pltpu.CompilerParams(use_tc_tiling_on_sc=False),
        scratch_types=[
            pltpu.VMEM_SHARED((NPAD, HP), jnp.float32),
            pltpu.VMEM((RING, C), jnp.int32),
            pltpu.VMEM((RING, C), jnp.int32),
        ] + [pltpu.VMEM((C, HP), jnp.float32)] * NGB + [
            pltpu.VMEM((_ZROWS, HP), jnp.float32),
            pltpu.SemaphoreType.DMA,
        ] + [pltpu.SemaphoreType.DMA] * NGB + [
            pltpu.SemaphoreType.DMA,
        ],
    )
    return k(u, src3, dst3)


def _row_mask():
    rows = lax.broadcasted_iota(jnp.int32, (NPAD, 1), 0)
    return rows < N


def _fin_body(degp_ref, dinv_ref):
    p = degp_ref[...]
    deg = p[0] + p[1] + 1.0
    dinv_ref[...] = lax.rsqrt(jnp.maximum(deg, 1.0))


def _fin_kernel(degp):
    degp3 = degp.reshape(NC, NPAD // 128, 128)
    out = pl.pallas_call(
        _fin_body,
        out_shape=jax.ShapeDtypeStruct((NPAD // 128, 128), jnp.float32),
    )(degp3)
    return out.reshape(NPAD, 1)


def _ka_body(x_ref, w_ref, dinv_ref, u_ref):
    h = jnp.dot(x_ref[...], w_ref[...], preferred_element_type=jnp.float32)
    u_ref[...] = h * dinv_ref[...]


def _ka_kernel(x_pad, W1p, dinv_col):
    return pl.pallas_call(
        _ka_body,
        out_shape=jax.ShapeDtypeStruct((NPAD, HP), jnp.float32),
    )(x_pad, W1p, dinv_col)


def _bn_relu_masked(agg, gamma, beta, mask):
    aggm = jnp.where(mask, agg, 0.0)
    mu = jnp.sum(aggm, axis=0, keepdims=True) * (1.0 / N)
    cen = jnp.where(mask, agg - mu, 0.0)
    var = jnp.sum(cen * cen, axis=0, keepdims=True) * (1.0 / N)
    z = cen * lax.rsqrt(var + 1e-5) * gamma + beta
    return jnp.where(mask, jnp.maximum(z, 0.0), 0.0)


def _kb_body(p_ref, u_ref, dinv_ref, b_ref, g_ref, be_ref, wn_ref, un_ref):
    p = p_ref[...]
    mask = _row_mask()
    agg = dinv_ref[...] * (p[0] + p[1] + u_ref[...]) + b_ref[...]
    z = _bn_relu_masked(agg, g_ref[...], be_ref[...], mask)
    un_ref[...] = jnp.dot(z, wn_ref[...],
                          preferred_element_type=jnp.float32) * dinv_ref[...]


def _kb_kernel(p, u, dinv_col, bp, gp, bep, Wnp):
    return pl.pallas_call(
        _kb_body,
        out_shape=jax.ShapeDtypeStruct((NPAD, HP), jnp.float32),
    )(p, u, dinv_col, bp, gp, bep, Wnp)


_POOL_CHUNK = 1024


def _kc_body(p_ref, u_ref, dinv_ref, b_ref, g_ref, be_ref, batch_ref,
             fw1_ref, fb1_ref, fw2_ref, fb2_ref, out_ref):
    p = p_ref[...]
    mask = _row_mask()
    agg = dinv_ref[...] * (p[0] + p[1] + u_ref[...]) + b_ref[...]
    z = _bn_relu_masked(agg, g_ref[...], be_ref[...], mask)

    batch = batch_ref[...]
    sums = jnp.zeros((G, HP), jnp.float32)
    cnt = jnp.zeros((G, 1), jnp.float32)
    dn = (((0,), (0,)), ((), ()))
    ids = lax.broadcasted_iota(jnp.int32, (_POOL_CHUNK, G), 1)
    ones_col = jnp.ones((_POOL_CHUNK, 1), jnp.float32)
    for i in range(NPAD // _POOL_CHUNK):
        zc = z[i * _POOL_CHUNK:(i + 1) * _POOL_CHUNK]
        bc = batch[i * _POOL_CHUNK:(i + 1) * _POOL_CHUNK]
        onehot = (ids == bc).astype(jnp.float32)
        sums = sums + lax.dot_general(onehot, zc, dn,
                                      preferred_element_type=jnp.float32)
        cnt = cnt + lax.dot_general(onehot, ones_col, dn,
                                    preferred_element_type=jnp.float32)
    pooled = sums / jnp.maximum(cnt, 1.0)
    hfc = jnp.maximum(
        jnp.dot(pooled, fw1_ref[...], preferred_element_type=jnp.float32)
        + fb1_ref[...], 0.0)
    out_ref[...] = (jnp.dot(hfc, fw2_ref[...],
                            preferred_element_type=jnp.float32)
                    + fb2_ref[...])


def _kc_kernel(p, u, dinv_col, bp, gp, bep, batch_pad, fW1p, fb1, fW2, fb2):
    out = pl.pallas_call(
        _kc_body,
        out_shape=jax.ShapeDtypeStruct((G, 1), jnp.float32),
    )(p, u, dinv_col, bp, gp, bep,
      batch_pad, fW1p, fb1.reshape(1, 32), fW2, fb2.reshape(1, 1))
    return out.reshape(G)


def _pad_cols(a):
    return jnp.pad(a, ((0, 0), (0, HP - H)))


def kernel(x, edge_index, batch, W1, b1, W2, b2, W3, b3,
           g1, be1, g2, be2, g3, be3, fW1, fb1, fW2, fb2):
    pad_e = N + jnp.arange(EPAD - E, dtype=jnp.int32) % (NPAD - N)
    src3 = jnp.concatenate([edge_index[0], pad_e]).reshape(NW, NCHUNK, C)
    dst3 = jnp.concatenate([edge_index[1], pad_e]).reshape(NW, NCHUNK, C)
    x_pad = jnp.pad(x, ((0, NPAD - N), (0, 0)))
    batch_pad = jnp.pad(batch, (0, NPAD - N),
                        constant_values=G).reshape(NPAD, 1)

    W1p = _pad_cols(W1)
    W2p = _pad_cols(jnp.pad(W2, ((0, HP - H), (0, 0))))
    W3p = _pad_cols(jnp.pad(W3, ((0, HP - H), (0, 0))))
    fW1p = jnp.pad(fW1, ((0, HP - H), (0, 0)))
    b1p = _pad_cols(b1.reshape(1, H))
    b2p = _pad_cols(b2.reshape(1, H))
    b3p = _pad_cols(b3.reshape(1, H))
    g1p = _pad_cols(g1.reshape(1, H))
    g2p = _pad_cols(g2.reshape(1, H))
    g3p = _pad_cols(g3.reshape(1, H))
    be1p = _pad_cols(be1.reshape(1, H))
    be2p = _pad_cols(be2.reshape(1, H))
    be3p = _pad_cols(be3.reshape(1, H))

    degp = _deg_kernel(dst3)
    dinv_col = _fin_kernel(degp)
    u1 = _ka_kernel(x_pad, W1p, dinv_col)
    p1 = _mp_kernel(u1, src3, dst3)
    u2 = _kb_kernel(p1, u1, dinv_col, b1p, g1p, be1p, W2p)
    p2 = _mp_kernel(u2, src3, dst3)
    u3 = _kb_kernel(p2, u2, dinv_col, b2p, g2p, be2p, W3p)
    p3 = _mp_kernel(u3, src3, dst3)
    return _kc_kernel(p3, u3, dinv_col, b3p, g3p, be3p, batch_pad,
                      fW1p, fb1, fW2, fb2)


# speedup vs baseline: 1.0869x; 1.0077x over previous
import jax
import jax.numpy as jnp
from jax import lax
from jax.experimental import pallas as pl
from jax.experimental.pallas import tpu as pltpu
from jax.experimental.pallas import tpu_sc as plsc

N = 10000
D = 128
H = 64
HP = 64
E = 320000
G = 512

NC = 2
NS = 16
NW = NC * NS

C = 128
EPAD = 327680
EPT = EPAD // NW
NCHUNK = EPT // C
BLK = 8
NBLK = NCHUNK // BLK
RING = 2 * BLK
NGB = 8

NPAD = 10240
ROWS_PT = NPAD // NS

_MESH = dict(core_axis_name="c", subcore_axis_name="s",
             num_cores=NC, num_subcores=NS)


def _deg_body(dst_hbm, out_hbm, acc, dst_v, ones_v, zbuf, sem_d):
    c = lax.axis_index("c")
    s = lax.axis_index("s")
    wid = c * NS + s
    zero16 = jnp.zeros((16,), jnp.float32)
    one16 = jnp.ones((16,), jnp.float32)
    for k in range(ROWS_PT // 16):
        zbuf[pl.ds(16 * k, 16)] = zero16
    for k in range(C // 16):
        ones_v[pl.ds(16 * k, 16)] = one16
    pltpu.sync_copy(zbuf, acc.at[pl.ds(s * ROWS_PT, ROWS_PT)])
    pltpu.sync_copy(dst_hbm.at[wid], dst_v)
    plsc.subcore_barrier()

    _DEG_WIN = 20
    for j in range(NCHUNK):
        pltpu.async_copy(ones_v, acc.at[dst_v.at[j]], sem_d, add=True)
        if j >= _DEG_WIN:
            pltpu.make_async_copy(
                ones_v, acc.at[dst_v.at[j - _DEG_WIN]], sem_d).wait()
    for j in range(NCHUNK - _DEG_WIN, NCHUNK):
        pltpu.make_async_copy(ones_v, acc.at[dst_v.at[j]], sem_d).wait()
    plsc.subcore_barrier()
    pltpu.sync_copy(acc.at[pl.ds(s * ROWS_PT, ROWS_PT)],
                    out_hbm.at[c, pl.ds(s * ROWS_PT, ROWS_PT)])


def _deg_kernel(dst3):
    k = pl.kernel(
        _deg_body,
        out_type=jax.ShapeDtypeStruct((NC, NPAD), jnp.float32),
        mesh=plsc.VectorSubcoreMesh(**_MESH),
        compiler_params=pltpu.CompilerParams(use_tc_tiling_on_sc=False),
        scratch_types=[
            pltpu.VMEM_SHARED((NPAD,), jnp.float32),
            pltpu.VMEM((NCHUNK, C), jnp.int32),
            pltpu.VMEM((C,), jnp.float32),
            pltpu.VMEM((ROWS_PT,), jnp.float32),
            pltpu.SemaphoreType.DMA,
        ],
    )
    return k(dst3)


_ZROWS = 32


def _mp_body(u_hbm, src_hbm, dst_hbm, out_hbm,
             acc, srcb, dstb, gbufs, zbuf, sem_i, gsems, ssems, semz):
    c = lax.axis_index("c")
    s = lax.axis_index("s")
    wid = c * NS + s

    zero16 = jnp.zeros((16,), jnp.float32)
    for i in range(_ZROWS):
        for k in range(HP // 16):
            zbuf[i, pl.ds(16 * k, 16)] = zero16
    for t in range(ROWS_PT // _ZROWS):
        pltpu.async_copy(
            zbuf, acc.at[pl.ds(s * ROWS_PT + t * _ZROWS, _ZROWS)], semz)
    for t in range(ROWS_PT // _ZROWS):
        pltpu.make_async_copy(
            zbuf, acc.at[pl.ds(s * ROWS_PT + t * _ZROWS, _ZROWS)],
            semz).wait()
    plsc.subcore_barrier()

    def prefetch_blk(b):
        slot = lax.rem(b, 2) * BLK
        pltpu.async_copy(src_hbm.at[wid, pl.ds(b * BLK, BLK)],
                         srcb.at[pl.ds(slot, BLK)], sem_i)
        pltpu.async_copy(dst_hbm.at[wid, pl.ds(b * BLK, BLK)],
                         dstb.at[pl.ds(slot, BLK)], sem_i)

    def wait_blk(b):
        slot = lax.rem(b, 2) * BLK
        pltpu.make_async_copy(src_hbm.at[wid, pl.ds(b * BLK, BLK)],
                              srcb.at[pl.ds(slot, BLK)], sem_i).wait()
        pltpu.make_async_copy(dst_hbm.at[wid, pl.ds(b * BLK, BLK)],
                              dstb.at[pl.ds(slot, BLK)], sem_i).wait()

    def scatter_wait(k):
        pltpu.make_async_copy(
            gbufs[k], acc.at[dstb.at[jnp.int32(0)]], ssems[k]).wait()

    def fire_gather(j, k, first=False):
        if not first:
            @pl.when(j >= NGB)
            def _():
                scatter_wait(k)
        pltpu.async_copy(u_hbm.at[srcb.at[lax.rem(j, RING)]],
                         gbufs[k], gsems[k])

    def drain_scatter(j, k):
        pltpu.make_async_copy(
            u_hbm.at[srcb.at[lax.rem(j, RING)]], gbufs[k], gsems[k]).wait()
        pltpu.async_copy(gbufs[k], acc.at[dstb.at[lax.rem(j, RING)]],
                         ssems[k], add=True)

    prefetch_blk(jnp.int32(0))
    prefetch_blk(jnp.int32(1))
    wait_blk(jnp.int32(0))
    for k in range(NGB - 1):
        fire_gather(jnp.int32(k), k, first=True)

    def body(b, carry):
        j = b * BLK

        @pl.when(b + 1 < NBLK)
        def _():
            wait_blk(b + 1)

        for jo in range(BLK):
            @pl.when(j + jo + NGB - 1 < NCHUNK)
            def _():
                fire_gather(j + jo + NGB - 1, (jo + NGB - 1) % NGB)

            drain_scatter(j + jo, jo % NGB)

        @pl.when(b + 2 < NBLK)
        def _():
            prefetch_blk(b + 2)

        return carry

    lax.fori_loop(0, NBLK, body, 0)
    for k in range(NGB):
        scatter_wait(k)
    plsc.subcore_barrier()
    pltpu.sync_copy(acc.at[pl.ds(s * ROWS_PT, ROWS_PT)],
                    out_hbm.at[c, pl.ds(s * ROWS_PT, ROWS_PT)])


def _mp_kernel(u, src3, dst3):
    def body(u_hbm, src_hbm, dst_hbm, out_hbm, acc, srcb, dstb,
             g0, g1, g2, g3, g4, g5, g6, g7, zbuf, sem_i,
             s0, s1, s2, s3, s4, s5, s6, s7,
             t0, t1, t2, t3, t4, t5, t6, t7, semz):
        _mp_body(u_hbm, src_hbm, dst_hbm, out_hbm, acc, srcb, dstb,
                 [g0, g1, g2, g3, g4, g5, g6, g7], zbuf, sem_i,
                 [s0, s1, s2, s3, s4, s5, s6, s7],
                 [t0, t1, t2, t3, t4, t5, t6, t7], semz)

    k = pl.kernel(
        body,
        out_type=jax.ShapeDtypeStruct((NC, NPAD, HP), jnp.float32),
        mesh=plsc.VectorSubcoreMesh(**_MESH),
        compiler_params=pltpu.CompilerParams(use_tc_tiling_on_sc=False),
        scratch_types=[
            pltpu.VMEM_SHARED((NPAD, HP), jnp.float32),
            pltpu.VMEM((RING, C), jnp.int32),
            pltpu.VMEM((RING, C), jnp.int32),
        ] + [pltpu.VMEM((C, HP), jnp.float32)] * NGB + [
            pltpu.VMEM((_ZROWS, HP), jnp.float32),
            pltpu.SemaphoreType.DMA,
        ] + [pltpu.SemaphoreType.DMA] * (2 * NGB) + [
            pltpu.SemaphoreType.DMA,
        ],
    )
    return k(u, src3, dst3)


def _row_mask():
    rows = lax.broadcasted_iota(jnp.int32, (NPAD, 1), 0)
    return rows < N


def _fin_body(degp_ref, dinv_ref):
    p = degp_ref[...]
    deg = p[0] + p[1] + 1.0
    dinv_ref[...] = lax.rsqrt(jnp.maximum(deg, 1.0))


def _fin_kernel(degp):
    degp3 = degp.reshape(NC, NPAD // 128, 128)
    out = pl.pallas_call(
        _fin_body,
        out_shape=jax.ShapeDtypeStruct((NPAD // 128, 128), jnp.float32),
    )(degp3)
    return out.reshape(NPAD, 1)


def _ka_body(x_ref, w_ref, dinv_ref, u_ref):
    h = jnp.dot(x_ref[...], w_ref[...], preferred_element_type=jnp.float32)
    u_ref[...] = h * dinv_ref[...]


def _ka_kernel(x_pad, W1p, dinv_col):
    return pl.pallas_call(
        _ka_body,
        out_shape=jax.ShapeDtypeStruct((NPAD, HP), jnp.float32),
    )(x_pad, W1p, dinv_col)


def _bn_relu_masked(agg, gamma, beta, mask):
    aggm = jnp.where(mask, agg, 0.0)
    mu = jnp.sum(aggm, axis=0, keepdims=True) * (1.0 / N)
    cen = jnp.where(mask, agg - mu, 0.0)
    var = jnp.sum(cen * cen, axis=0, keepdims=True) * (1.0 / N)
    z = cen * lax.rsqrt(var + 1e-5) * gamma + beta
    return jnp.where(mask, jnp.maximum(z, 0.0), 0.0)


def _kb_body(p_ref, u_ref, dinv_ref, b_ref, g_ref, be_ref, wn_ref, un_ref):
    p = p_ref[...]
    mask = _row_mask()
    agg = dinv_ref[...] * (p[0] + p[1] + u_ref[...]) + b_ref[...]
    z = _bn_relu_masked(agg, g_ref[...], be_ref[...], mask)
    un_ref[...] = jnp.dot(z, wn_ref[...],
                          preferred_element_type=jnp.float32) * dinv_ref[...]


def _kb_kernel(p, u, dinv_col, bp, gp, bep, Wnp):
    return pl.pallas_call(
        _kb_body,
        out_shape=jax.ShapeDtypeStruct((NPAD, HP), jnp.float32),
    )(p, u, dinv_col, bp, gp, bep, Wnp)


_POOL_CHUNK = 1024


def _kc_body(p_ref, u_ref, dinv_ref, b_ref, g_ref, be_ref, batch_ref,
             fw1_ref, fb1_ref, fw2_ref, fb2_ref, out_ref):
    p = p_ref[...]
    mask = _row_mask()
    agg = dinv_ref[...] * (p[0] + p[1] + u_ref[...]) + b_ref[...]
    z = _bn_relu_masked(agg, g_ref[...], be_ref[...], mask)

    batch = batch_ref[...]
    sums = jnp.zeros((G, HP), jnp.float32)
    cnt = jnp.zeros((G, 1), jnp.float32)
    dn = (((0,), (0,)), ((), ()))
    ids = lax.broadcasted_iota(jnp.int32, (_POOL_CHUNK, G), 1)
    ones_col = jnp.ones((_POOL_CHUNK, 1), jnp.float32)
    for i in range(NPAD // _POOL_CHUNK):
        zc = z[i * _POOL_CHUNK:(i + 1) * _POOL_CHUNK]
        bc = batch[i * _POOL_CHUNK:(i + 1) * _POOL_CHUNK]
        onehot = (ids == bc).astype(jnp.float32)
        sums = sums + lax.dot_general(onehot, zc, dn,
                                      preferred_element_type=jnp.float32)
        cnt = cnt + lax.dot_general(onehot, ones_col, dn,
                                    preferred_element_type=jnp.float32)
    pooled = sums / jnp.maximum(cnt, 1.0)
    hfc = jnp.maximum(
        jnp.dot(pooled, fw1_ref[...], preferred_element_type=jnp.float32)
        + fb1_ref[...], 0.0)
    out_ref[...] = (jnp.dot(hfc, fw2_ref[...],
                            preferred_element_type=jnp.float32)
                    + fb2_ref[...])


def _kc_kernel(p, u, dinv_col, bp, gp, bep, batch_pad, fW1p, fb1, fW2, fb2):
    out = pl.pallas_call(
        _kc_body,
        out_shape=jax.ShapeDtypeStruct((G, 1), jnp.float32),
    )(p, u, dinv_col, bp, gp, bep,
      batch_pad, fW1p, fb1.reshape(1, 32), fW2, fb2.reshape(1, 1))
    return out.reshape(G)


def _pad_cols(a):
    return jnp.pad(a, ((0, 0), (0, HP - H)))


def kernel(x, edge_index, batch, W1, b1, W2, b2, W3, b3,
           g1, be1, g2, be2, g3, be3, fW1, fb1, fW2, fb2):
    pad_e = N + jnp.arange(EPAD - E, dtype=jnp.int32) % (NPAD - N)
    src3 = jnp.concatenate([edge_index[0], pad_e]).reshape(NW, NCHUNK, C)
    dst3 = jnp.concatenate([edge_index[1], pad_e]).reshape(NW, NCHUNK, C)
    x_pad = jnp.pad(x, ((0, NPAD - N), (0, 0)))
    batch_pad = jnp.pad(batch, (0, NPAD - N),
                        constant_values=G).reshape(NPAD, 1)

    W1p = _pad_cols(W1)
    W2p = _pad_cols(jnp.pad(W2, ((0, HP - H), (0, 0))))
    W3p = _pad_cols(jnp.pad(W3, ((0, HP - H), (0, 0))))
    fW1p = jnp.pad(fW1, ((0, HP - H), (0, 0)))
    b1p = _pad_cols(b1.reshape(1, H))
    b2p = _pad_cols(b2.reshape(1, H))
    b3p = _pad_cols(b3.reshape(1, H))
    g1p = _pad_cols(g1.reshape(1, H))
    g2p = _pad_cols(g2.reshape(1, H))
    g3p = _pad_cols(g3.reshape(1, H))
    be1p = _pad_cols(be1.reshape(1, H))
    be2p = _pad_cols(be2.reshape(1, H))
    be3p = _pad_cols(be3.reshape(1, H))

    degp = _deg_kernel(dst3)
    dinv_col = _fin_kernel(degp)
    u1 = _ka_kernel(x_pad, W1p, dinv_col)
    p1 = _mp_kernel(u1, src3, dst3)
    u2 = _kb_kernel(p1, u1, dinv_col, b1p, g1p, be1p, W2p)
    p2 = _mp_kernel(u2, src3, dst3)
    u3 = _kb_kernel(p2, u2, dinv_col, b2p, g2p, be2p, W3p)
    p3 = _mp_kernel(u3, src3, dst3)
    return _kc_kernel(p3, u3, dinv_col, b3p, g3p, be3p, batch_pad,
                      fW1p, fb1, fW2, fb2)
